# baseline (device time: 33506 ns/iter reference)
import jax
import jax.numpy as jnp
from jax import lax
from jax.experimental import pallas as pl
from jax.experimental.pallas import tpu as pltpu

N_DEV = 32
BLK = 64

_sem_signal = getattr(pl, "semaphore_signal", None) or pltpu.semaphore_signal
_sem_wait = getattr(pl, "semaphore_wait", None) or pltpu.semaphore_wait
_DevIdType = getattr(pl, "DeviceIdType", None) or pltpu.DeviceIdType


def kernel(x, w_mat):
    k_global, k_shard = x.shape
    _, n = w_mat.shape
    assert k_shard == BLK and k_global == N_DEV * BLK

    def body(x_ref, w_ref, out_ref, xbf_ref, stack_ref, xg_ref, wbf_ref,
             send_sems, recv_sems, ack_sem):
        my = lax.axis_index("i")

        xbf_ref[...] = x_ref[...].astype(jnp.bfloat16)

        def send_desc(k):
            tgt = lax.rem(my + k, N_DEV)
            return pltpu.make_async_remote_copy(
                src_ref=xbf_ref.at[pl.ds(tgt * BLK, BLK), :],
                dst_ref=stack_ref.at[my],
                send_sem=send_sems.at[k],
                recv_sem=recv_sems.at[my],
                device_id=(tgt,),
                device_id_type=_DevIdType.MESH,
            )

        for k in range(N_DEV):
            send_desc(k).start()

        wbf_ref[...] = w_ref[...].astype(jnp.bfloat16)

        for j in range(N_DEV):
            recv = pltpu.make_async_remote_copy(
                src_ref=xbf_ref.at[pl.ds(0, BLK), :],
                dst_ref=stack_ref.at[j],
                send_sem=send_sems.at[0],
                recv_sem=recv_sems.at[j],
                device_id=(0,),
                device_id_type=_DevIdType.MESH,
            )
            recv.wait_recv()
            xg_ref[:, j * BLK:(j + 1) * BLK] = stack_ref[j]

        for k in range(1, N_DEV):
            _sem_signal(
                ack_sem, 1,
                device_id=(lax.rem(my + k, N_DEV),),
                device_id_type=_DevIdType.MESH,
            )

        y = jnp.dot(xg_ref[...], wbf_ref[...],
                    preferred_element_type=jnp.float32)
        out_ref[...] = jnp.maximum(y, 0.0)

        for k in range(N_DEV):
            send_desc(k).wait_send()
        _sem_wait(ack_sem, N_DEV - 1)

    return pl.pallas_call(
        body,
        out_shape=jax.ShapeDtypeStruct((BLK, n), jnp.float32),
        in_specs=[
            pl.BlockSpec(memory_space=pltpu.VMEM),
            pl.BlockSpec(memory_space=pltpu.VMEM),
        ],
        out_specs=pl.BlockSpec(memory_space=pltpu.VMEM),
        scratch_shapes=[
            pltpu.VMEM((k_global, BLK), jnp.bfloat16),
            pltpu.VMEM((N_DEV, BLK, BLK), jnp.bfloat16),
            pltpu.VMEM((BLK, k_global), jnp.bfloat16),
            pltpu.VMEM((k_global, n), jnp.bfloat16),
            pltpu.SemaphoreType.DMA((N_DEV,)),
            pltpu.SemaphoreType.DMA((N_DEV,)),
            pltpu.SemaphoreType.REGULAR,
        ],
    )(x, w_mat)


# device time: 28719 ns/iter; 1.1667x vs baseline; 1.1667x over previous
import os

import jax
import jax.numpy as jnp
from jax import lax
from jax.experimental import pallas as pl
from jax.experimental.pallas import tpu as pltpu

N_DEV = 32
BLK = 64
N_CHUNKS = 8

_ABLATE = set(os.environ.get("KERNEL_ABLATE", "").split(","))
_PEERS = int(os.environ.get("KERNEL_PEERS", str(N_DEV - 1)))
_BARRIER = os.environ.get("KERNEL_BARRIER", "ring")

_MS = getattr(pltpu, "MemorySpace", None) or getattr(pltpu, "TPUMemorySpace")
_ANY = getattr(pl, "ANY", None) or _MS.ANY
_sem_signal = getattr(pl, "semaphore_signal", None) or pltpu.semaphore_signal
_sem_wait = getattr(pl, "semaphore_wait", None) or pltpu.semaphore_wait
_DevIdType = getattr(pl, "DeviceIdType", None) or pltpu.DeviceIdType


def kernel(x, w_mat):
    k_global, k_shard = x.shape
    _, n = w_mat.shape
    assert k_shard == BLK and k_global == N_DEV * BLK
    cw = n // N_CHUNKS

    def body(x_ref, w_hbm, out_ref, xbf_ref, stack_ref, xg_ref, wf_ref,
             send_sems, recv_sems, wdma_sems, ack_sem):
        my = lax.axis_index("i")

        if _BARRIER != "auto":
            nbrs = (
                [lax.rem(my + 1, N_DEV), lax.rem(my + N_DEV - 1, N_DEV)]
                if _BARRIER == "ring"
                else [lax.rem(my + k, N_DEV) for k in range(1, N_DEV)]
            )
            barrier_sem = pltpu.get_barrier_semaphore()
            for nbr in nbrs:
                _sem_signal(barrier_sem, 1, device_id=(nbr,),
                            device_id_type=_DevIdType.MESH)
            _sem_wait(barrier_sem, len(nbrs))

        xbf_ref[...] = x_ref[...].astype(jnp.bfloat16)

        def send_desc(k):
            tgt = lax.rem(my + k, N_DEV)
            return pltpu.make_async_remote_copy(
                src_ref=xbf_ref.at[pl.ds(tgt * BLK, BLK), :],
                dst_ref=stack_ref.at[my],
                send_sem=send_sems.at[k],
                recv_sem=recv_sems.at[my],
                device_id=(tgt,),
                device_id_type=_DevIdType.MESH,
            )

        comm = "nocomm" not in _ABLATE
        sends = [0] + list(range(1, _PEERS + 1))
        if comm:
            for k in sends:
                send_desc(k).start()

        def wdma(c):
            return pltpu.make_async_copy(
                w_hbm.at[:, pl.ds(c * cw, cw)],
                wf_ref.at[:, pl.ds(c * cw, cw)],
                wdma_sems.at[c],
            )

        if "nowdma" not in _ABLATE:
            for c in range(N_CHUNKS):
                wdma(c).start()

        for k in sends if comm else []:
            src = lax.rem(my + N_DEV - k, N_DEV)
            recv = pltpu.make_async_remote_copy(
                src_ref=xbf_ref.at[pl.ds(0, BLK), :],
                dst_ref=stack_ref.at[src],
                send_sem=send_sems.at[0],
                recv_sem=recv_sems.at[src],
                device_id=(0,),
                device_id_type=_DevIdType.MESH,
            )
            recv.wait_recv()

        if "noassemble" not in _ABLATE:
            for j in range(N_DEV):
                xg_ref[:, j * BLK:(j + 1) * BLK] = stack_ref[j]

        if comm:
            for k in sends[1:]:
                _sem_signal(
                    ack_sem, 1,
                    device_id=(lax.rem(my + N_DEV - k, N_DEV),),
                    device_id_type=_DevIdType.MESH,
                )

        if "nogemm" not in _ABLATE:
            for c in range(N_CHUNKS):
                wdma(c).wait()
                wbf = wf_ref[:, c * cw:(c + 1) * cw].astype(jnp.bfloat16)
                y = jnp.dot(xg_ref[...], wbf,
                            preferred_element_type=jnp.float32)
                out_ref[:, c * cw:(c + 1) * cw] = jnp.maximum(y, 0.0)
        else:
            if "nowdma" not in _ABLATE:
                for c in range(N_CHUNKS):
                    wdma(c).wait()
            out_ref[...] = jnp.zeros_like(out_ref)

        if comm:
            for k in sends:
                send_desc(k).wait_send()
            _sem_wait(ack_sem, len(sends) - 1)

    return pl.pallas_call(
        body,
        out_shape=jax.ShapeDtypeStruct((BLK, n), jnp.float32),
        in_specs=[
            pl.BlockSpec(memory_space=pltpu.VMEM),
            pl.BlockSpec(memory_space=_ANY),
        ],
        out_specs=pl.BlockSpec(memory_space=pltpu.VMEM),
        scratch_shapes=[
            pltpu.VMEM((k_global, BLK), jnp.bfloat16),
            pltpu.VMEM((N_DEV, BLK, BLK), jnp.bfloat16),
            pltpu.VMEM((BLK, k_global), jnp.bfloat16),
            pltpu.VMEM((k_global, n), jnp.float32),
            pltpu.SemaphoreType.DMA((N_DEV,)),
            pltpu.SemaphoreType.DMA((N_DEV,)),
            pltpu.SemaphoreType.DMA((N_CHUNKS,)),
            pltpu.SemaphoreType.REGULAR,
        ],
        compiler_params=(
            pltpu.CompilerParams(collective_id=0)
            if _BARRIER != "auto" else pltpu.CompilerParams()
        ),
    )(x, w_mat)


# device time: 22973 ns/iter; 1.4585x vs baseline; 1.2501x over previous
import os

import jax
import jax.numpy as jnp
from jax import lax
from jax.experimental import pallas as pl
from jax.experimental.pallas import tpu as pltpu

N_DEV = 32
BLK = 64
N_CHUNKS = 8

_ABLATE = set(os.environ.get("KERNEL_ABLATE", "").split(","))
_PEERS = int(os.environ.get("KERNEL_PEERS", str(N_DEV - 1)))
_BARRIER = os.environ.get("KERNEL_BARRIER", "ring")

_MS = getattr(pltpu, "MemorySpace", None) or getattr(pltpu, "TPUMemorySpace")
_ANY = getattr(_MS, "HBM", None) or getattr(pl, "ANY")
_sem_signal = getattr(pl, "semaphore_signal", None) or pltpu.semaphore_signal
_sem_wait = getattr(pl, "semaphore_wait", None) or pltpu.semaphore_wait
_DevIdType = getattr(pl, "DeviceIdType", None) or pltpu.DeviceIdType


def kernel(x, w_mat):
    k_global, k_shard = x.shape
    _, n = w_mat.shape
    assert k_shard == BLK and k_global == N_DEV * BLK
    cw = n // N_CHUNKS

    def body(x_ref, w_hbm, out_ref, xbf_ref, stack_ref, xg_ref, wf_ref,
             send_sems, recv_sems, wdma_sems, ack_sem):
        my = lax.axis_index("i")

        if _BARRIER != "auto":
            nbrs = (
                [lax.rem(my + 1, N_DEV), lax.rem(my + N_DEV - 1, N_DEV)]
                if _BARRIER == "ring"
                else [lax.rem(my + k, N_DEV) for k in range(1, N_DEV)]
            )
            barrier_sem = pltpu.get_barrier_semaphore()
            for nbr in nbrs:
                _sem_signal(barrier_sem, 1, device_id=(nbr,),
                            device_id_type=_DevIdType.MESH)
            _sem_wait(barrier_sem, len(nbrs))

        xbf_ref[...] = x_ref[...].astype(jnp.bfloat16)

        def send_desc(k):
            tgt = lax.rem(my + k, N_DEV)
            return pltpu.make_async_remote_copy(
                src_ref=xbf_ref.at[pl.ds(tgt * BLK, BLK), :],
                dst_ref=stack_ref.at[my],
                send_sem=send_sems.at[k],
                recv_sem=recv_sems.at[my],
                device_id=(tgt,),
                device_id_type=_DevIdType.MESH,
            )

        comm = "nocomm" not in _ABLATE
        sends = [0] + list(range(1, _PEERS + 1))
        if comm:
            for k in sends:
                send_desc(k).start()

        def wdma(c):
            return pltpu.make_async_copy(
                w_hbm.at[:, pl.ds(c * cw, cw)],
                wf_ref.at[:, pl.ds(c * cw, cw)],
                wdma_sems.at[c],
            )

        if "nowdma" not in _ABLATE:
            for c in range(N_CHUNKS):
                wdma(c).start()

        for k in sends if comm else []:
            src = lax.rem(my + N_DEV - k, N_DEV)
            recv = pltpu.make_async_remote_copy(
                src_ref=xbf_ref.at[pl.ds(0, BLK), :],
                dst_ref=stack_ref.at[src],
                send_sem=send_sems.at[0],
                recv_sem=recv_sems.at[src],
                device_id=(0,),
                device_id_type=_DevIdType.MESH,
            )
            recv.wait_recv()

        if "noassemble" not in _ABLATE:
            for j in range(N_DEV):
                xg_ref[:, j * BLK:(j + 1) * BLK] = stack_ref[j]

        if comm:
            for k in sends[1:]:
                _sem_signal(
                    ack_sem, 1,
                    device_id=(lax.rem(my + N_DEV - k, N_DEV),),
                    device_id_type=_DevIdType.MESH,
                )

        if "nogemm" not in _ABLATE:
            for c in range(N_CHUNKS):
                wdma(c).wait()
                wbf = wf_ref[:, c * cw:(c + 1) * cw].astype(jnp.bfloat16)
                y = jnp.dot(xg_ref[...], wbf,
                            preferred_element_type=jnp.float32)
                out_ref[:, c * cw:(c + 1) * cw] = jnp.maximum(y, 0.0)
        else:
            if "nowdma" not in _ABLATE:
                for c in range(N_CHUNKS):
                    wdma(c).wait()
            out_ref[...] = jnp.zeros_like(out_ref)

        if comm:
            for k in sends:
                send_desc(k).wait_send()
            _sem_wait(ack_sem, len(sends) - 1)

    return pl.pallas_call(
        body,
        out_shape=jax.ShapeDtypeStruct((BLK, n), jnp.float32),
        in_specs=[
            pl.BlockSpec(memory_space=pltpu.VMEM),
            pl.BlockSpec(memory_space=_ANY),
        ],
        out_specs=pl.BlockSpec(memory_space=pltpu.VMEM),
        scratch_shapes=[
            pltpu.VMEM((k_global, BLK), jnp.bfloat16),
            pltpu.VMEM((N_DEV, BLK, BLK), jnp.bfloat16),
            pltpu.VMEM((BLK, k_global), jnp.bfloat16),
            pltpu.VMEM((k_global, n), jnp.float32),
            pltpu.SemaphoreType.DMA((N_DEV,)),
            pltpu.SemaphoreType.DMA((N_DEV,)),
            pltpu.SemaphoreType.DMA((N_CHUNKS,)),
            pltpu.SemaphoreType.REGULAR,
        ],
        compiler_params=(
            pltpu.CompilerParams(collective_id=0)
            if _BARRIER != "auto" else pltpu.CompilerParams()
        ),
    )(x, pltpu.with_memory_space_constraint(w_mat, _ANY))


# device time: 22819 ns/iter; 1.4683x vs baseline; 1.0067x over previous
import os

import jax
import jax.numpy as jnp
from jax import lax
from jax.experimental import pallas as pl
from jax.experimental.pallas import tpu as pltpu

N_DEV = 32
BLK = 64
N_Q = 4
PER_Q = N_DEV // N_Q

_ABLATE = set(os.environ.get("KERNEL_ABLATE", "").split(","))
_BARRIER = os.environ.get("KERNEL_BARRIER", "ring")

_MS = getattr(pltpu, "MemorySpace", None) or getattr(pltpu, "TPUMemorySpace")
_ANY = getattr(_MS, "HBM", None) or getattr(pl, "ANY")
_sem_signal = getattr(pl, "semaphore_signal", None) or pltpu.semaphore_signal
_sem_wait = getattr(pl, "semaphore_wait", None) or pltpu.semaphore_wait
_DevIdType = getattr(pl, "DeviceIdType", None) or pltpu.DeviceIdType


def kernel(x, w_mat):
    k_global, k_shard = x.shape
    _, n = w_mat.shape
    assert k_shard == BLK and k_global == N_DEV * BLK
    kq = k_global // N_Q

    def body(x_ref, w_hbm, out_hbm, xbf_ref, stack_ref, xg_ref, wf_ref,
             acc_ref, yout_ref, send_sems, recv_sems, wdma_sems, out_sem,
             ack_sem):
        my = lax.axis_index("i")

        if _BARRIER != "auto":
            nbrs = (
                [lax.rem(my + 1, N_DEV), lax.rem(my + N_DEV - 1, N_DEV)]
                if _BARRIER == "ring"
                else [lax.rem(my + k, N_DEV) for k in range(1, N_DEV)]
            )
            barrier_sem = pltpu.get_barrier_semaphore()
            for nbr in nbrs:
                _sem_signal(barrier_sem, 1, device_id=(nbr,),
                            device_id_type=_DevIdType.MESH)
            _sem_wait(barrier_sem, len(nbrs))

        xbf_ref[...] = x_ref[...].astype(jnp.bfloat16)

        def send_desc(k):
            tgt = lax.rem(my + k, N_DEV)
            return pltpu.make_async_remote_copy(
                src_ref=xbf_ref.at[pl.ds(tgt * BLK, BLK), :],
                dst_ref=stack_ref.at[my],
                send_sem=send_sems.at[k],
                recv_sem=recv_sems.at[my],
                device_id=(tgt,),
                device_id_type=_DevIdType.MESH,
            )

        for k in range(N_DEV):
            send_desc(k).start()

        def wdma(q):
            return pltpu.make_async_copy(
                w_hbm.at[pl.ds(q * kq, kq), :],
                wf_ref.at[pl.ds(q * kq, kq), :],
                wdma_sems.at[q],
            )

        for q in range(N_Q):
            wdma(q).start()

        for q in range(N_Q):
            for j in range(q * PER_Q, (q + 1) * PER_Q):
                recv = pltpu.make_async_remote_copy(
                    src_ref=xbf_ref.at[pl.ds(0, BLK), :],
                    dst_ref=stack_ref.at[j],
                    send_sem=send_sems.at[0],
                    recv_sem=recv_sems.at[j],
                    device_id=(0,),
                    device_id_type=_DevIdType.MESH,
                )
                recv.wait_recv()
                xg_ref[:, j * BLK:(j + 1) * BLK] = stack_ref[j]
            for j in range(q * PER_Q, (q + 1) * PER_Q):
                @pl.when(j != my)
                def _():
                    _sem_signal(ack_sem, 1, device_id=(j,),
                                device_id_type=_DevIdType.MESH)

            wdma(q).wait()
            yq = jnp.dot(
                xg_ref[:, q * kq:(q + 1) * kq],
                wf_ref[q * kq:(q + 1) * kq, :].astype(jnp.bfloat16),
                preferred_element_type=jnp.float32,
            )
            if q == 0:
                acc_ref[...] = yq
            elif q < N_Q - 1:
                acc_ref[...] += yq
            else:
                yout_ref[...] = jnp.maximum(acc_ref[...] + yq, 0.0)

        out_copy = pltpu.make_async_copy(yout_ref, out_hbm, out_sem)
        out_copy.start()

        for k in range(N_DEV):
            send_desc(k).wait_send()
        _sem_wait(ack_sem, N_DEV - 1)
        out_copy.wait()

    return pl.pallas_call(
        body,
        out_shape=jax.ShapeDtypeStruct((BLK, n), jnp.float32),
        in_specs=[
            pl.BlockSpec(memory_space=pltpu.VMEM),
            pl.BlockSpec(memory_space=_ANY),
        ],
        out_specs=pl.BlockSpec(memory_space=_ANY),
        scratch_shapes=[
            pltpu.VMEM((k_global, BLK), jnp.bfloat16),
            pltpu.VMEM((N_DEV, BLK, BLK), jnp.bfloat16),
            pltpu.VMEM((BLK, k_global), jnp.bfloat16),
            pltpu.VMEM((k_global, n), jnp.float32),
            pltpu.VMEM((BLK, n), jnp.float32),
            pltpu.VMEM((BLK, n), jnp.float32),
            pltpu.SemaphoreType.DMA((N_DEV,)),
            pltpu.SemaphoreType.DMA((N_DEV,)),
            pltpu.SemaphoreType.DMA((N_Q,)),
            pltpu.SemaphoreType.DMA,
            pltpu.SemaphoreType.REGULAR,
        ],
        compiler_params=(
            pltpu.CompilerParams(collective_id=0)
            if _BARRIER != "auto" else pltpu.CompilerParams()
        ),
    )(x, pltpu.with_memory_space_constraint(w_mat, _ANY))


# device time: 20798 ns/iter; 1.6110x vs baseline; 1.0972x over previous
import os

import jax
import jax.numpy as jnp
from jax import lax
from jax.experimental import pallas as pl
from jax.experimental.pallas import tpu as pltpu

N_DEV = 32
BLK = 64
N_Q = 4
PER_Q = N_DEV // N_Q
LOG2_N = 5

_ABLATE = set(os.environ.get("KERNEL_ABLATE", "").split(","))

_MS = getattr(pltpu, "MemorySpace", None) or getattr(pltpu, "TPUMemorySpace")
_ANY = getattr(_MS, "HBM", None) or getattr(pl, "ANY")
_sem_signal = getattr(pl, "semaphore_signal", None) or pltpu.semaphore_signal
_sem_wait = getattr(pl, "semaphore_wait", None) or pltpu.semaphore_wait
_DevIdType = getattr(pl, "DeviceIdType", None) or pltpu.DeviceIdType


def kernel(x, w_mat):
    k_global, k_shard = x.shape
    _, n = w_mat.shape
    assert k_shard == BLK and k_global == N_DEV * BLK
    kq = k_global // N_Q

    def body(x_ref, w_hbm, out_ref, xbf_ref, stack_ref, xg_ref, wf_ref,
             acc_ref, send_sem, recv_qsems, wdma_sems, round_sems):
        my = lax.axis_index("i")
        myq = my // PER_Q

        barrier_sem = pltpu.get_barrier_semaphore()
        _sem_signal(barrier_sem, 1)
        _sem_wait(barrier_sem, 1)

        xbf_ref[...] = x_ref[...].astype(jnp.bfloat16)

        def send_desc(k):
            tgt = lax.rem(my + k, N_DEV)
            return pltpu.make_async_remote_copy(
                src_ref=xbf_ref.at[pl.ds(tgt * BLK, BLK), :],
                dst_ref=stack_ref.at[my],
                send_sem=send_sem,
                recv_sem=recv_qsems.at[myq],
                device_id=(tgt,),
                device_id_type=_DevIdType.MESH,
            )

        for k in range(N_DEV):
            send_desc(k).start()

        def wdma(q):
            return pltpu.make_async_copy(
                w_hbm.at[pl.ds(q * kq, kq), :],
                wf_ref.at[pl.ds(q * kq, kq), :],
                wdma_sems.at[q],
            )

        for q in range(N_Q):
            wdma(q).start()

        for q in range(N_Q):
            qsl = stack_ref.at[pl.ds(q * PER_Q, PER_Q)]
            pltpu.make_async_remote_copy(
                src_ref=qsl,
                dst_ref=qsl,
                send_sem=send_sem,
                recv_sem=recv_qsems.at[q],
                device_id=(0,),
                device_id_type=_DevIdType.MESH,
            ).wait_recv()
            for j in range(q * PER_Q, (q + 1) * PER_Q):
                xg_ref[:, j * BLK:(j + 1) * BLK] = stack_ref[j]
            wdma(q).wait()
            yq = jnp.dot(
                xg_ref[:, q * kq:(q + 1) * kq],
                wf_ref[q * kq:(q + 1) * kq, :].astype(jnp.bfloat16),
                preferred_element_type=jnp.float32,
            )
            if q == 0:
                acc_ref[...] = yq
            elif q < N_Q - 1:
                acc_ref[...] += yq
            else:
                out_ref[...] = jnp.maximum(acc_ref[...] + yq, 0.0)

        pltpu.make_async_remote_copy(
            src_ref=xbf_ref,
            dst_ref=xbf_ref,
            send_sem=send_sem,
            recv_sem=recv_qsems.at[0],
            device_id=(0,),
            device_id_type=_DevIdType.MESH,
        ).wait_send()

        if "nobarrier" not in _ABLATE:
            for r in range(LOG2_N):
                _sem_signal(round_sems.at[r], 1,
                            device_id=(lax.rem(my + (1 << r), N_DEV),),
                            device_id_type=_DevIdType.MESH)
                _sem_wait(round_sems.at[r], 1)

    return pl.pallas_call(
        body,
        out_shape=jax.ShapeDtypeStruct((BLK, n), jnp.float32),
        in_specs=[
            pl.BlockSpec(memory_space=pltpu.VMEM),
            pl.BlockSpec(memory_space=_ANY),
        ],
        out_specs=pl.BlockSpec(memory_space=pltpu.VMEM),
        scratch_shapes=[
            pltpu.VMEM((k_global, BLK), jnp.bfloat16),
            pltpu.VMEM((N_DEV, BLK, BLK), jnp.bfloat16),
            pltpu.VMEM((BLK, k_global), jnp.bfloat16),
            pltpu.VMEM((k_global, n), jnp.float32),
            pltpu.VMEM((BLK, n), jnp.float32),
            pltpu.SemaphoreType.DMA,
            pltpu.SemaphoreType.DMA((N_Q,)),
            pltpu.SemaphoreType.DMA((N_Q,)),
            pltpu.SemaphoreType.REGULAR((LOG2_N,)),
        ],
        compiler_params=pltpu.CompilerParams(collective_id=0),
    )(x, pltpu.with_memory_space_constraint(w_mat, _ANY))
